# baseline (speedup 1.0000x reference)
import jax
import jax.numpy as jnp
from jax import lax
from jax.experimental import pallas as pl
from jax.experimental.pallas import tpu as pltpu
from jax.experimental.pallas import tpu_sc as plsc

_B, _S, _D = 4, 2048, 1024
_L = 16
_NC, _NS = 2, 16
_NW = _NC * _NS
_ROWS = _B * _S
_RPW = _ROWS // _NW
_WPB = _S // _RPW
_SEQ_START = 4
_RB = 16
_NB = _RPW // _RB
_NBLK = _ROWS // _RB
_CPR = _D // _L
_NUM_SEQ = 2048
_AMAX = _NUM_SEQ // 8 - 3


def _sc_body(ids_hbm, ctrl_hbm, tab_hbm, st_hbm, meta_hbm, pool_hbm,
             ids_v, st_v, idx_v, val_v, ctrl_v, gbuf, meta_v, sem_g):
    cid = lax.axis_index("c")
    sid = lax.axis_index("s")
    wid = sid * _NC + cid
    b = wid // _WPB
    p0 = (wid % _WPB) * _RPW

    pltpu.sync_copy(st_hbm, st_v)
    st_vec = st_v[...]
    iota = lax.iota(jnp.int32, _L)

    _gdn = lax.GatherDimensionNumbers(offset_dims=(),
                                      collapsed_slice_dims=(0,),
                                      start_index_map=(0,))

    def _take16(v, idx):
        return lax.gather(v, idx[:, None], dimension_numbers=_gdn,
                          slice_sizes=(1,),
                          mode=lax.GatherScatterMode.PROMISE_IN_BOUNDS)

    def _cummax16(v):
        for k in (1, 2, 4, 8):
            v = jnp.maximum(v, _take16(v, jnp.maximum(iota - k, 0)))
        return v

    def _marked(k):
        v = ids_v[pl.ds(k * _L, _L)]
        pos = k * _L + iota
        return jnp.where((v == st_vec) & (pos >= _SEQ_START), pos,
                         jnp.int32(-1))

    pltpu.sync_copy(ids_hbm.at[b], ids_v)

    def pref_body(k, cv):
        return jnp.maximum(cv, _marked(k))

    carry_v = lax.fori_loop(0, p0 // _L, pref_body,
                            jnp.full((_L,), -1, jnp.int32))
    carry0 = _cummax16(carry_v)[15]

    def own_body(k, carry):
        m = _marked(p0 // _L + k)
        ls = jnp.maximum(_cummax16(m), carry)
        pos = p0 + k * _L + iota
        validm = ls >= 0
        idx_v[k, :] = jnp.where(validm, pos - ls, jnp.int32(0))
        val_v[pl.ds(k * _L, _L)] = jnp.where(validm, 1.0, 0.0)
        return ls[15]

    lax.fori_loop(0, _RPW // _L, own_body, carry0)

    @pl.when(p0 == 0)
    def _():
        pltpu.sync_copy(ctrl_hbm, ctrl_v)

    zero16 = jnp.zeros((_L,), jnp.int32)
    a_vec, r_vec, k_vec, p_vec = zero16, zero16, zero16, zero16
    a_prev = jnp.int32(0)
    p_prev = wid * _NB

    for k in range(_NB):
        idxb = idx_v[k, pl.ds(0, _L)]
        vmb = val_v[pl.ds(k * _L, _L)]
        validm = vmb > 0.5
        any_v = _cummax16(jnp.where(validm, 1, 0))[15] > 0
        ib0 = _take16(idxb, jnp.zeros((_L,), jnp.int32))
        okm = jnp.logical_and(validm, idxb == ib0 + iota)
        all_lin = _cummax16(jnp.where(okm, 0, 1))[15] == 0
        is_ctrl = (p0 == 0) & (k == 0)
        is_lin = jnp.logical_and(all_lin, jnp.logical_not(is_ctrl))
        rel0 = idxb[0]
        a_blk = jnp.minimum(rel0 // 8, _AMAX)
        roff = rel0 - a_blk * 8
        kind = jnp.where(is_lin, 1,
                         jnp.where(jnp.logical_or(any_v, is_ctrl), 0, 2))

        @pl.when(kind == 0)
        def _(k=k, vmb=vmb, is_ctrl=is_ctrl):
            cg = pltpu.async_copy(tab_hbm.at[idx_v.at[k]], gbuf, sem_g)
            cg.wait()

            def row_body(r, _):
                vm = _take16(vmb, jnp.full((_L,), r, jnp.int32))
                for c in range(_CPR):
                    g = gbuf[r, pl.ds(c * _L, _L)]
                    gbuf[r, pl.ds(c * _L, _L)] = g * vm
                return 0

            lax.fori_loop(0, _RB, row_body, 0)

            @pl.when(is_ctrl)
            def _():
                for r in range(_SEQ_START):
                    for c in range(_CPR):
                        gbuf[r, pl.ds(c * _L, _L)] = (
                            ctrl_v[r, pl.ds(c * _L, _L)])

            pltpu.sync_copy(gbuf, pool_hbm.at[wid * _NB + k])

        a_sc = jnp.where(kind == 1, a_blk, a_prev)
        r_sc = jnp.where(kind == 1, roff, jnp.int32(0))
        p_sc = jnp.where(kind == 0, wid * _NB + k, p_prev)
        a_prev, p_prev = a_sc, p_sc
        lane = iota == k
        a_vec = jnp.where(lane, a_sc, a_vec)
        r_vec = jnp.where(lane, r_sc, r_vec)
        k_vec = jnp.where(lane, kind, k_vec)
        p_vec = jnp.where(lane, p_sc, p_vec)

    meta_v[pl.ds(0, _L)] = a_vec
    meta_v[pl.ds(_L, _L)] = r_vec
    meta_v[pl.ds(2 * _L, _L)] = k_vec
    meta_v[pl.ds(3 * _L, _L)] = p_vec
    pltpu.sync_copy(meta_v, meta_hbm.at[pl.ds(wid * 4 * _L, 4 * _L)])


def _tc_body(s_ref, x_ref, s0_ref, s1_ref, s2_ref, pool_ref, out_ref):
    i = pl.program_id(0)
    base = (i // _NB) * 64 + (i % _NB)
    roff = s_ref[base + _L]
    kind = s_ref[base + 2 * _L]
    w = jnp.concatenate([s0_ref[...], s1_ref[...], s2_ref[...]], axis=0)
    e = w[0:_RB, :]
    for r in range(1, 8):
        e = jnp.where(roff == r, w[r:r + _RB, :], e)
    pe = jnp.where(kind == 1, e,
                   jnp.where(kind == 2, jnp.float32(0.0), pool_ref[0]))
    out_ref[...] = x_ref[...] + pe


def _meta_pos(i):
    return (i // _NB) * 64 + (i % _NB)


def kernel(x, input_ids, control_table, sequence_table, start_token):
    x2 = x.reshape(_ROWS, _D)
    ids = input_ids.astype(jnp.int32)
    st = jnp.full((_L,), start_token, jnp.int32)
    mesh = plsc.VectorSubcoreMesh(core_axis_name="c", subcore_axis_name="s",
                                  num_cores=_NC, num_subcores=_NS)
    meta, pool = pl.kernel(
        _sc_body,
        out_type=(jax.ShapeDtypeStruct((_NW * 4 * _L,), jnp.int32),
                  jax.ShapeDtypeStruct((_NBLK, _RB, _D), jnp.float32)),
        mesh=mesh,
        scratch_types=[
            pltpu.VMEM((_S,), jnp.int32),
            pltpu.VMEM((_L,), jnp.int32),
            pltpu.VMEM((_NB, _RB), jnp.int32),
            pltpu.VMEM((_RPW,), jnp.float32),
            pltpu.VMEM((_SEQ_START, _D), jnp.float32),
            pltpu.VMEM((_RB, _D), jnp.float32),
            pltpu.VMEM((4 * _L,), jnp.int32),
            pltpu.SemaphoreType.DMA,
        ],
    )(ids, control_table, sequence_table, st)

    grid_spec = pltpu.PrefetchScalarGridSpec(
        num_scalar_prefetch=1,
        grid=(_NBLK,),
        in_specs=[
            pl.BlockSpec((_RB, _D), lambda i, s: (i, 0)),
            pl.BlockSpec((8, _D), lambda i, s: (s[_meta_pos(i)], 0)),
            pl.BlockSpec((8, _D), lambda i, s: (s[_meta_pos(i)] + 1, 0)),
            pl.BlockSpec((8, _D), lambda i, s: (s[_meta_pos(i)] + 2, 0)),
            pl.BlockSpec((1, _RB, _D),
                         lambda i, s: (s[_meta_pos(i) + 3 * _L], 0, 0)),
        ],
        out_specs=pl.BlockSpec((_RB, _D), lambda i, s: (i, 0)),
    )
    out = pl.pallas_call(
        _tc_body,
        grid_spec=grid_spec,
        out_shape=jax.ShapeDtypeStruct((_ROWS, _D), jnp.float32),
    )(meta, x2, sequence_table, sequence_table, sequence_table, pool)
    return out.reshape(_B, _S, _D)
